# Initial kernel scaffold; baseline (speedup 1.0000x reference)
import functools
import math

import jax
import jax.numpy as jnp
from jax import lax
from jax.experimental import pallas as pl
from jax.experimental.pallas import tpu as pltpu

_B, _T, _C = 1, 2048, 1024
_H, _HD = 16, 64
_E, _TOPK, _FF = 8, 2, 4096
_HALF = _HD // 2
_SQRT2 = math.sqrt(2.0)


def _layernorm(x, w, b, eps=1e-5, clip=65000.0):
    mu = jnp.mean(x, axis=-1, keepdims=True)
    xc = x - mu
    var = jnp.mean(xc * xc, axis=-1, keepdims=True)
    y = xc * lax.rsqrt(var + eps)
    y = jnp.clip(y, -clip, clip)
    return y * w + b


def _gelu_exact(x):
    return 0.5 * x * (1.0 + lax.erf(x / _SQRT2))


def _ln_body(x_ref, w_ref, b_ref, o_ref):
    o_ref[...] = _layernorm(x_ref[...], w_ref[...], b_ref[...])


def _ln_call(x, w, b):
    return pl.pallas_call(
        _ln_body,
        out_shape=jax.ShapeDtypeStruct((_T, _C), jnp.float32),
    )(x, w.reshape(1, _C), b.reshape(1, _C))


def _qkv_body(ln_ref, wq_ref, wk_ref, wv_ref, sin_ref, cos_ref,
              q_ref, k_ref, v_ref):
    x = ln_ref[...]
    dn = (((1,), (1,)), ((), ()))
    q = lax.dot_general(x, wq_ref[...], dn, preferred_element_type=jnp.float32)
    k = lax.dot_general(x, wk_ref[...], dn, preferred_element_type=jnp.float32)
    v = lax.dot_general(x, wv_ref[...], dn, preferred_element_type=jnp.float32)
    sin = sin_ref[...]
    cos = cos_ref[...]

    def rope(t):
        t1 = t[:, :_HALF]
        t2 = t[:, _HALF:]
        return jnp.concatenate([t1 * cos - t2 * sin, t1 * sin + t2 * cos],
                               axis=-1)

    q_ref[0] = rope(q)
    k_ref[0] = rope(k)
    v_ref[0] = v


def _qkv_call(ln1, qkv_w, sin, cos):
    return pl.pallas_call(
        _qkv_body,
        grid=(_H,),
        in_specs=[
            pl.BlockSpec((_T, _C), lambda h: (0, 0)),
            pl.BlockSpec((_HD, _C), lambda h: (3 * h, 0)),
            pl.BlockSpec((_HD, _C), lambda h: (3 * h + 1, 0)),
            pl.BlockSpec((_HD, _C), lambda h: (3 * h + 2, 0)),
            pl.BlockSpec((_T, _HALF), lambda h: (0, 0)),
            pl.BlockSpec((_T, _HALF), lambda h: (0, 0)),
        ],
        out_specs=[
            pl.BlockSpec((1, _T, _HD), lambda h: (h, 0, 0)),
            pl.BlockSpec((1, _T, _HD), lambda h: (h, 0, 0)),
            pl.BlockSpec((1, _T, _HD), lambda h: (h, 0, 0)),
        ],
        out_shape=[jax.ShapeDtypeStruct((_H, _T, _HD), jnp.float32)] * 3,
    )(ln1, qkv_w, qkv_w, qkv_w, sin, cos)


_BQ = 256
_BK = 256


def _attn_body(q_ref, k_ref, v_ref, o_ref):
    qb = pl.program_id(1)
    q = q_ref[0]
    scale = 1.0 / math.sqrt(_HD)
    dn = (((1,), (1,)), ((), ()))

    def body(i, carry):
        acc, m, l = carry
        k = k_ref[0][pl.ds(i * _BK, _BK), :]
        v = v_ref[0][pl.ds(i * _BK, _BK), :]
        s = lax.dot_general(q, k, dn, preferred_element_type=jnp.float32)
        s = s * scale
        col = i * _BK + lax.broadcasted_iota(jnp.int32, (_BQ, _BK), 1)
        row = qb * _BQ + lax.broadcasted_iota(jnp.int32, (_BQ, _BK), 0)
        s = jnp.where(col <= row, s, -1e30)
        m_new = jnp.maximum(m, jnp.max(s, axis=-1, keepdims=True))
        p = jnp.exp(s - m_new)
        alpha = jnp.exp(m - m_new)
        l = l * alpha + jnp.sum(p, axis=-1, keepdims=True)
        acc = acc * alpha + jnp.dot(p, v, preferred_element_type=jnp.float32)
        return acc, m_new, l

    nk = (qb + 1) * (_BQ // _BK)
    acc, m, l = lax.fori_loop(
        0, nk, body,
        (jnp.zeros((_BQ, _HD), jnp.float32),
         jnp.full((_BQ, 1), -1e38, jnp.float32),
         jnp.zeros((_BQ, 1), jnp.float32)))
    o_ref[0] = acc / l


def _attn_call(q, k, v):
    return pl.pallas_call(
        _attn_body,
        grid=(_H, _T // _BQ),
        in_specs=[
            pl.BlockSpec((1, _BQ, _HD), lambda h, qb: (h, qb, 0)),
            pl.BlockSpec((1, _T, _HD), lambda h, qb: (h, 0, 0)),
            pl.BlockSpec((1, _T, _HD), lambda h, qb: (h, 0, 0)),
        ],
        out_specs=pl.BlockSpec((1, _BQ, _HD), lambda h, qb: (h, qb, 0)),
        out_shape=jax.ShapeDtypeStruct((_H, _T, _HD), jnp.float32),
    )(q, k, v)


def _proj_body(ctx_ref, pw_ref, x_ref, w2_ref, b2_ref, gw_ref,
               x1_ref, h2_ref, wt_ref, acc_ref):
    h = pl.program_id(0)

    @pl.when(h == 0)
    def _():
        acc_ref[...] = jnp.zeros_like(acc_ref)

    dn = (((1,), (1,)), ((), ()))
    acc_ref[...] += lax.dot_general(ctx_ref[0], pw_ref[...], dn,
                                    preferred_element_type=jnp.float32)

    @pl.when(h == _H - 1)
    def _():
        x1 = x_ref[...] + acc_ref[...]
        x1_ref[...] = x1
        h2 = _layernorm(x1, w2_ref[...], b2_ref[...])
        h2_ref[...] = h2
        logits = lax.dot_general(h2, gw_ref[...], dn,
                                 preferred_element_type=jnp.float32)
        mx = jnp.max(logits, axis=-1, keepdims=True)
        p = jnp.exp(logits - mx)
        g = p / jnp.sum(p, axis=-1, keepdims=True)
        ii = lax.broadcasted_iota(jnp.int32, (_T, _E), 1)
        m1 = jnp.max(g, axis=-1, keepdims=True)
        i1 = jnp.min(jnp.where(g == m1, ii, _E), axis=-1, keepdims=True)
        sel1 = ii == i1
        g2 = jnp.where(sel1, -1.0, g)
        m2 = jnp.max(g2, axis=-1, keepdims=True)
        i2 = jnp.min(jnp.where(g2 == m2, ii, _E), axis=-1, keepdims=True)
        sel2 = ii == i2
        wt = jnp.where(sel1, m1, 0.0) + jnp.where(sel2, m2, 0.0)
        wt_ref[...] = wt


def _proj_call(ctx, proj_w, x, ln2_w, ln2_b, gate_w):
    return pl.pallas_call(
        _proj_body,
        grid=(_H,),
        in_specs=[
            pl.BlockSpec((1, _T, _HD), lambda h: (h, 0, 0)),
            pl.BlockSpec((_C, _HD), lambda h: (0, h)),
            pl.BlockSpec((_T, _C), lambda h: (0, 0)),
            pl.BlockSpec((1, _C), lambda h: (0, 0)),
            pl.BlockSpec((1, _C), lambda h: (0, 0)),
            pl.BlockSpec((_E, _C), lambda h: (0, 0)),
        ],
        out_specs=[
            pl.BlockSpec((_T, _C), lambda h: (0, 0)),
            pl.BlockSpec((_T, _C), lambda h: (0, 0)),
            pl.BlockSpec((_T, _E), lambda h: (0, 0)),
        ],
        out_shape=[
            jax.ShapeDtypeStruct((_T, _C), jnp.float32),
            jax.ShapeDtypeStruct((_T, _C), jnp.float32),
            jax.ShapeDtypeStruct((_T, _E), jnp.float32),
        ],
        scratch_shapes=[pltpu.VMEM((_T, _C), jnp.float32)],
    )(ctx, proj_w, x, ln2_w.reshape(1, _C), ln2_b.reshape(1, _C), gate_w)


_FT = 1024


def _moe_body(h2_ref, w1_ref, b1_ref, w2_ref, b2_ref, wt_ref, x1_ref,
              o_ref, acc_ref):
    e = pl.program_id(0)
    f = pl.program_id(1)

    @pl.when(jnp.logical_and(e == 0, f == 0))
    def _():
        acc_ref[...] = jnp.zeros_like(acc_ref)

    dn = (((1,), (1,)), ((), ()))
    hf = lax.dot_general(h2_ref[...], w1_ref[0], dn,
                         preferred_element_type=jnp.float32)
    hf = _gelu_exact(hf + b1_ref[0])
    o = lax.dot_general(hf, w2_ref[0], dn,
                        preferred_element_type=jnp.float32)
    onehot = (lax.broadcasted_iota(jnp.int32, (_E, 1), 0) == e
              ).astype(jnp.float32)
    we = jnp.dot(wt_ref[...], onehot,
                 preferred_element_type=jnp.float32)
    acc_ref[...] += we * o

    @pl.when(f == 0)
    def _():
        acc_ref[...] += we * b2_ref[0]

    @pl.when(jnp.logical_and(e == _E - 1, f == _FF // _FT - 1))
    def _():
        o_ref[...] = x1_ref[...] + acc_ref[...]


def _moe_call(h2, w1, b1, w2, b2, wt, x1):
    return pl.pallas_call(
        _moe_body,
        grid=(_E, _FF // _FT),
        in_specs=[
            pl.BlockSpec((_T, _C), lambda e, f: (0, 0)),
            pl.BlockSpec((1, _FT, _C), lambda e, f: (e, f, 0)),
            pl.BlockSpec((1, _FT), lambda e, f: (e, f)),
            pl.BlockSpec((1, _C, _FT), lambda e, f: (e, 0, f)),
            pl.BlockSpec((1, _C), lambda e, f: (e, 0)),
            pl.BlockSpec((_T, _E), lambda e, f: (0, 0)),
            pl.BlockSpec((_T, _C), lambda e, f: (0, 0)),
        ],
        out_specs=pl.BlockSpec((_T, _C), lambda e, f: (0, 0)),
        out_shape=jax.ShapeDtypeStruct((_T, _C), jnp.float32),
        scratch_shapes=[pltpu.VMEM((_T, _C), jnp.float32)],
    )(h2, w1, b1, w2, b2, wt, x1)


def kernel(x, ln1_w, ln1_b, ln2_w, ln2_b, qkv_w, proj_w, gate_w, w1, b1,
           w2, b2):
    x2d = x.reshape(_T, _C)
    pos = jnp.arange(_T, dtype=jnp.float32)[:, None]
    inv_freq = 1.0 / (10000.0 ** (
        jnp.arange(0, _HD, 2, dtype=jnp.float32) / _HD))
    ang = pos * inv_freq
    sin = jnp.sin(ang)
    cos = jnp.cos(ang)

    ln1 = _ln_call(x2d, ln1_w, ln1_b)
    q, k, v = _qkv_call(ln1, qkv_w, sin, cos)
    ctx = _attn_call(q, k, v)
    x1, h2, wt = _proj_call(ctx, proj_w, x2d, ln2_w, ln2_b, gate_w)
    out = _moe_call(h2, w1, b1, w2, b2, wt, x1)
    return out.reshape(_B, _T, _C)


# speedup vs baseline: 1.8410x; 1.8410x over previous
import functools
import math

import jax
import jax.numpy as jnp
from jax import lax
from jax.experimental import pallas as pl
from jax.experimental.pallas import tpu as pltpu

_B, _T, _C = 1, 2048, 1024
_H, _HD = 16, 64
_E, _TOPK, _FF = 8, 2, 4096
_HALF = _HD // 2
_SQRT2 = math.sqrt(2.0)


def _layernorm(x, w, b, eps=1e-5, clip=65000.0):
    mu = jnp.mean(x, axis=-1, keepdims=True)
    xc = x - mu
    var = jnp.mean(xc * xc, axis=-1, keepdims=True)
    y = xc * lax.rsqrt(var + eps)
    y = jnp.clip(y, -clip, clip)
    return y * w + b


def _gelu_exact(x):
    return 0.5 * x * (1.0 + lax.erf(x / _SQRT2))


def _ln_body(x_ref, w_ref, b_ref, o_ref):
    o_ref[...] = _layernorm(x_ref[...], w_ref[...], b_ref[...])


def _ln_call(x, w, b):
    return pl.pallas_call(
        _ln_body,
        out_shape=jax.ShapeDtypeStruct((_T, _C), jnp.float32),
    )(x, w.reshape(1, _C), b.reshape(1, _C))


def _qkv_body(ln_ref, wq_ref, wk_ref, wv_ref, sin_ref, cos_ref,
              q_ref, k_ref, v_ref):
    x = ln_ref[...]
    dn = (((1,), (1,)), ((), ()))
    q = lax.dot_general(x, wq_ref[...], dn, preferred_element_type=jnp.float32)
    k = lax.dot_general(x, wk_ref[...], dn, preferred_element_type=jnp.float32)
    v = lax.dot_general(x, wv_ref[...], dn, preferred_element_type=jnp.float32)
    sin = sin_ref[...]
    cos = cos_ref[...]

    def rope(t):
        t1 = t[:, :_HALF]
        t2 = t[:, _HALF:]
        return jnp.concatenate([t1 * cos - t2 * sin, t1 * sin + t2 * cos],
                               axis=-1)

    q_ref[0] = rope(q)
    k_ref[0] = rope(k)
    v_ref[0] = v


def _qkv_call(ln1, qkv_w, sin, cos):
    return pl.pallas_call(
        _qkv_body,
        grid=(_H,),
        in_specs=[
            pl.BlockSpec((_T, _C), lambda h: (0, 0)),
            pl.BlockSpec((_HD, _C), lambda h: (3 * h, 0)),
            pl.BlockSpec((_HD, _C), lambda h: (3 * h + 1, 0)),
            pl.BlockSpec((_HD, _C), lambda h: (3 * h + 2, 0)),
            pl.BlockSpec((_T, _HALF), lambda h: (0, 0)),
            pl.BlockSpec((_T, _HALF), lambda h: (0, 0)),
        ],
        out_specs=[
            pl.BlockSpec((1, _T, _HD), lambda h: (h, 0, 0)),
            pl.BlockSpec((1, _T, _HD), lambda h: (h, 0, 0)),
            pl.BlockSpec((1, _T, _HD), lambda h: (h, 0, 0)),
        ],
        out_shape=[jax.ShapeDtypeStruct((_H, _T, _HD), jnp.float32)] * 3,
    )(ln1, qkv_w, qkv_w, qkv_w, sin, cos)


_BQ = 256
_BK = 256


def _attn_body(q_ref, k_ref, v_ref, o_ref):
    qb = pl.program_id(1)
    q = q_ref[0]
    scale = 1.0 / math.sqrt(_HD)
    dn = (((1,), (1,)), ((), ()))

    def body(i, carry):
        acc, m, l = carry
        k = k_ref[0, pl.ds(i * _BK, _BK), :]
        v = v_ref[0, pl.ds(i * _BK, _BK), :]
        s = lax.dot_general(q, k, dn, preferred_element_type=jnp.float32)
        s = s * scale
        col = i * _BK + lax.broadcasted_iota(jnp.int32, (_BQ, _BK), 1)
        row = qb * _BQ + lax.broadcasted_iota(jnp.int32, (_BQ, _BK), 0)
        s = jnp.where(col <= row, s, -1e30)
        m_new = jnp.maximum(m, jnp.max(s, axis=-1, keepdims=True))
        p = jnp.exp(s - m_new)
        alpha = jnp.exp(m - m_new)
        l = l * alpha + jnp.sum(p, axis=-1, keepdims=True)
        acc = acc * alpha + jnp.dot(p, v, preferred_element_type=jnp.float32)
        return acc, m_new, l

    nk = (qb + 1) * (_BQ // _BK)
    acc, m, l = lax.fori_loop(
        0, nk, body,
        (jnp.zeros((_BQ, _HD), jnp.float32),
         jnp.full((_BQ, 1), -1e38, jnp.float32),
         jnp.zeros((_BQ, 1), jnp.float32)))
    o_ref[0] = acc / l


def _attn_call(q, k, v):
    return pl.pallas_call(
        _attn_body,
        grid=(_H, _T // _BQ),
        in_specs=[
            pl.BlockSpec((1, _BQ, _HD), lambda h, qb: (h, qb, 0)),
            pl.BlockSpec((1, _T, _HD), lambda h, qb: (h, 0, 0)),
            pl.BlockSpec((1, _T, _HD), lambda h, qb: (h, 0, 0)),
        ],
        out_specs=pl.BlockSpec((1, _BQ, _HD), lambda h, qb: (h, qb, 0)),
        out_shape=jax.ShapeDtypeStruct((_H, _T, _HD), jnp.float32),
    )(q, k, v)


def _proj_body(ctx_ref, pw_ref, x_ref, w2_ref, b2_ref, gw_ref,
               x1_ref, h2_ref, wt_ref, acc_ref):
    h = pl.program_id(0)

    @pl.when(h == 0)
    def _():
        acc_ref[...] = jnp.zeros_like(acc_ref)

    dn = (((1,), (1,)), ((), ()))
    acc_ref[...] += lax.dot_general(ctx_ref[0], pw_ref[0], dn,
                                    preferred_element_type=jnp.float32)

    @pl.when(h == _H - 1)
    def _():
        x1 = x_ref[...] + acc_ref[...]
        x1_ref[...] = x1
        h2 = _layernorm(x1, w2_ref[...], b2_ref[...])
        h2_ref[...] = h2
        logits = lax.dot_general(h2, gw_ref[...], dn,
                                 preferred_element_type=jnp.float32)
        mx = jnp.max(logits, axis=-1, keepdims=True)
        p = jnp.exp(logits - mx)
        g = p / jnp.sum(p, axis=-1, keepdims=True)
        ii = lax.broadcasted_iota(jnp.int32, (_T, _E), 1)
        m1 = jnp.max(g, axis=-1, keepdims=True)
        i1 = jnp.min(jnp.where(g == m1, ii, _E), axis=-1, keepdims=True)
        sel1 = ii == i1
        g2 = jnp.where(sel1, -1.0, g)
        m2 = jnp.max(g2, axis=-1, keepdims=True)
        i2 = jnp.min(jnp.where(g2 == m2, ii, _E), axis=-1, keepdims=True)
        sel2 = ii == i2
        wt = jnp.where(sel1, m1, 0.0) + jnp.where(sel2, m2, 0.0)
        wt_ref[...] = wt


def _proj_call(ctx, proj_w, x, ln2_w, ln2_b, gate_w):
    return pl.pallas_call(
        _proj_body,
        grid=(_H,),
        in_specs=[
            pl.BlockSpec((1, _T, _HD), lambda h: (h, 0, 0)),
            pl.BlockSpec((1, _C, _HD), lambda h: (h, 0, 0)),
            pl.BlockSpec((_T, _C), lambda h: (0, 0)),
            pl.BlockSpec((1, _C), lambda h: (0, 0)),
            pl.BlockSpec((1, _C), lambda h: (0, 0)),
            pl.BlockSpec((_E, _C), lambda h: (0, 0)),
        ],
        out_specs=[
            pl.BlockSpec((_T, _C), lambda h: (0, 0)),
            pl.BlockSpec((_T, _C), lambda h: (0, 0)),
            pl.BlockSpec((_T, _E), lambda h: (0, 0)),
        ],
        out_shape=[
            jax.ShapeDtypeStruct((_T, _C), jnp.float32),
            jax.ShapeDtypeStruct((_T, _C), jnp.float32),
            jax.ShapeDtypeStruct((_T, _E), jnp.float32),
        ],
        scratch_shapes=[pltpu.VMEM((_T, _C), jnp.float32)],
    )(ctx, proj_w.reshape(_C, _H, _HD).transpose(1, 0, 2), x,
      ln2_w.reshape(1, _C), ln2_b.reshape(1, _C), gate_w)


_FT = 512


def _moe_body(h2_ref, w1_ref, b1_ref, w2_ref, b2_ref, wt_ref, x1_ref,
              o_ref, acc_ref):
    e = pl.program_id(0)
    f = pl.program_id(1)

    @pl.when(jnp.logical_and(e == 0, f == 0))
    def _():
        acc_ref[...] = jnp.zeros_like(acc_ref)

    dn = (((1,), (1,)), ((), ()))
    hf = lax.dot_general(h2_ref[...], w1_ref[0], dn,
                         preferred_element_type=jnp.float32)
    hf = _gelu_exact(hf + b1_ref[0, 0])
    o = lax.dot_general(hf, w2_ref[0], dn,
                        preferred_element_type=jnp.float32)
    onehot = (lax.broadcasted_iota(jnp.int32, (_E, 1), 0) == e
              ).astype(jnp.float32)
    we = jnp.dot(wt_ref[...], onehot,
                 preferred_element_type=jnp.float32)
    acc_ref[...] += we * o

    @pl.when(f == 0)
    def _():
        acc_ref[...] += we * b2_ref[0, 0]

    @pl.when(jnp.logical_and(e == _E - 1, f == _FF // _FT - 1))
    def _():
        o_ref[...] = x1_ref[...] + acc_ref[...]


def _moe_call(h2, w1, b1, w2, b2, wt, x1):
    return pl.pallas_call(
        _moe_body,
        grid=(_E, _FF // _FT),
        in_specs=[
            pl.BlockSpec((_T, _C), lambda e, f: (0, 0)),
            pl.BlockSpec((1, _FT, _C), lambda e, f: (e, f, 0)),
            pl.BlockSpec((1, 1, _FT), lambda e, f: (e, 0, f)),
            pl.BlockSpec((1, _C, _FT), lambda e, f: (e, 0, f)),
            pl.BlockSpec((1, 1, _C), lambda e, f: (e, 0, 0)),
            pl.BlockSpec((_T, _E), lambda e, f: (0, 0)),
            pl.BlockSpec((_T, _C), lambda e, f: (0, 0)),
        ],
        out_specs=pl.BlockSpec((_T, _C), lambda e, f: (0, 0)),
        out_shape=jax.ShapeDtypeStruct((_T, _C), jnp.float32),
        scratch_shapes=[pltpu.VMEM((_T, _C), jnp.float32)],
    )(h2, w1, b1.reshape(_E, 1, _FF), w2, b2.reshape(_E, 1, _C), wt, x1)


def kernel(x, ln1_w, ln1_b, ln2_w, ln2_b, qkv_w, proj_w, gate_w, w1, b1,
           w2, b2):
    x2d = x.reshape(_T, _C)
    pos = jnp.arange(_T, dtype=jnp.float32)[:, None]
    inv_freq = 1.0 / (10000.0 ** (
        jnp.arange(0, _HD, 2, dtype=jnp.float32) / _HD))
    ang = pos * inv_freq
    sin = jnp.sin(ang)
    cos = jnp.cos(ang)

    ln1 = _ln_call(x2d, ln1_w, ln1_b)
    q, k, v = _qkv_call(ln1, qkv_w, sin, cos)
    ctx = _attn_call(q, k, v)
    x1, h2, wt = _proj_call(ctx, proj_w, x2d, ln2_w, ln2_b, gate_w)
    out = _moe_call(h2, w1, b1, w2, b2, wt, x1)
    return out.reshape(_B, _T, _C)
